# baseline (speedup 1.0000x reference)
import functools
import math

import jax
import jax.numpy as jnp
from jax import lax
from jax.experimental import pallas as pl
from jax.experimental.pallas import tpu as pltpu
from jax.experimental.pallas import tpu_sc as plsc

N = 10000
E = 320000
H = 128
BLK = 2000
GRID = N // BLK

CHUNK = 64
G = 8
NW = 32
CPW = 320
NBLK = CPW // G
E2P = NW * CPW * CHUNK
NROW = E2P // CHUNK
NP = 10240
RPT = NP // 16


def _gelu(x):
    return 0.5 * x * (1.0 + lax.erf(x * (1.0 / math.sqrt(2.0))))


def _dense(h, w, b):
    return jnp.dot(h, w[...], preferred_element_type=jnp.float32) + b[...]


def _tc1_body(x_ref, al_ref, e0, e1, e2, pw1, pb1, pw2, pb2,
              w, b, wt, bt, ws, bs, h0_out, x2_out, s_out):
    x = x_ref[...]
    oh0 = (x[:, 0:1] == lax.broadcasted_iota(jnp.int32, (BLK, 16), 1))
    oh1 = (x[:, 1:2] == lax.broadcasted_iota(jnp.int32, (BLK, 8), 1))
    oh2 = (x[:, 2:3] == lax.broadcasted_iota(jnp.int32, (BLK, 4), 1))
    he0 = jnp.dot(oh0.astype(jnp.float32), e0[...],
                  preferred_element_type=jnp.float32)
    he1 = jnp.dot(oh1.astype(jnp.float32), e1[...],
                  preferred_element_type=jnp.float32)
    he2 = jnp.dot(oh2.astype(jnp.float32), e2[...],
                  preferred_element_type=jnp.float32)
    al = al_ref[...]
    k2 = lax.broadcasted_iota(jnp.int32, (1, 16), 1).astype(jnp.float32) * 2.0
    div = jnp.exp(k2 * (-math.log(10000.0) / 32.0))
    ang = al * div
    h = jnp.concatenate([he0, he1, he2, jnp.sin(ang), jnp.cos(ang)], axis=1)
    h = _gelu(_dense(h, pw1, pb1))
    h = _dense(h, pw2, pb2)
    h0_out[...] = h
    x2_out[0] = _dense(h, w, b)
    x2_out[1] = _dense(h, wt, bt)
    s_out[...] = _dense(h, ws, bs)


def _tc2_body(p_ref, s_ref, w, b, wt, bt, ws, bs, h_out, x2_out, s_out):
    h = _gelu(p_ref[0] + p_ref[1] + s_ref[...])
    h_out[...] = h
    x2_out[0] = _dense(h, w, b)
    x2_out[1] = _dense(h, wt, bt)
    s_out[...] = _dense(h, ws, bs)


def _tc3_body(p_ref, s_ref, h0_ref, h1_ref, pw1, pb1, pw2, pb2, out_ref):
    h2 = _gelu(p_ref[0] + p_ref[1] + s_ref[...])
    t = (jnp.dot(h0_ref[...], pw1[0:H], preferred_element_type=jnp.float32)
         + jnp.dot(h1_ref[...], pw1[H:2 * H], preferred_element_type=jnp.float32)
         + jnp.dot(h2, pw1[2 * H:3 * H], preferred_element_type=jnp.float32)
         + pb1[...])
    out_ref[...] = (jnp.dot(_gelu(t), pw2[...],
                            preferred_element_type=jnp.float32) + pb2[...])


def _row_spec():
    return pl.BlockSpec((BLK, H), lambda i: (i, 0))


def _full(shape):
    return pl.BlockSpec(shape, lambda i: tuple(0 for _ in shape))


_W = _full((H, H))
_B = _full((1, H))
_P2 = pl.BlockSpec((2, BLK, H), lambda i: (0, i, 0))
_X2_SPEC = pl.BlockSpec((2, BLK, H), lambda i: (0, i, 0))
_X2 = jax.ShapeDtypeStruct((2, N, H), jnp.float32)

_tc1 = pl.pallas_call(
    _tc1_body,
    grid=(GRID,),
    in_specs=[pl.BlockSpec((BLK, 3), lambda i: (i, 0)),
              pl.BlockSpec((BLK, 1), lambda i: (i, 0)),
              _full((16, 32)), _full((8, 32)), _full((4, 32)),
              _W, _B, _W, _B,
              _W, _B, _W, _B, _W, _B],
    out_specs=[_row_spec(), _X2_SPEC, _row_spec()],
    out_shape=[jax.ShapeDtypeStruct((N, H), jnp.float32), _X2,
               jax.ShapeDtypeStruct((N, H), jnp.float32)],
)

_tc2 = pl.pallas_call(
    _tc2_body,
    grid=(GRID,),
    in_specs=[_P2, _row_spec(),
              _W, _B, _W, _B, _W, _B],
    out_specs=[_row_spec(), _X2_SPEC, _row_spec()],
    out_shape=[jax.ShapeDtypeStruct((N, H), jnp.float32), _X2,
               jax.ShapeDtypeStruct((N, H), jnp.float32)],
)

_tc3 = pl.pallas_call(
    _tc3_body,
    grid=(GRID,),
    in_specs=[_P2, _row_spec(), _row_spec(), _row_spec(),
              _full((3 * H, H)), _B, _W, _B],
    out_specs=_row_spec(),
    out_shape=jax.ShapeDtypeStruct((N, H), jnp.float32),
)


def _sc_edge_body(x2_hbm, src_hbm, dst_hbm, out_hbm,
                  isrc, idst, buf0, buf1, buf2, buf3, acc,
                  sg0, sg1, sg2, sg3, ss0, ss1, ss2, ss3,
                  si0, si1, sj0, sj1):
    c = lax.axis_index("c")
    tid = lax.axis_index("s")
    wid = tid * 2 + c
    base = wid * CPW

    bufs = (buf0, buf1, buf2, buf3)
    sg = (sg0, sg1, sg2, sg3)
    ss = (ss0, ss1, ss2, ss3)
    si = (si0, si1)
    sj = (sj0, sj1)

    @pl.loop(0, CHUNK)
    def _zero_rows(i):
        for k in range(H // 16):
            buf0[i, pl.ds(k * 16, 16)] = jnp.zeros((16,), jnp.float32)

    for r in range(RPT // CHUNK):
        pltpu.sync_copy(buf0, acc.at[pl.ds((tid * (RPT // CHUNK) + r) * CHUNK,
                                           CHUNK)])
    plsc.subcore_barrier()

    def stage(g, p, sync=False):
        rows = pl.ds(base + g * G, G)
        if sync:
            pltpu.sync_copy(src_hbm.at[rows], isrc.at[p])
            pltpu.sync_copy(dst_hbm.at[rows], idst.at[p])
        else:
            pltpu.async_copy(src_hbm.at[rows], isrc.at[p], si[p])
            pltpu.async_copy(dst_hbm.at[rows], idst.at[p], sj[p])

    def wait_stage(g, p):
        rows = pl.ds(base + g * G, G)
        pltpu.make_async_copy(src_hbm.at[rows], isrc.at[p], si[p]).wait()
        pltpu.make_async_copy(dst_hbm.at[rows], idst.at[p], sj[p]).wait()

    def per_parity(g1, fn):
        par = lax.rem(g1, 2)

        @pl.when(par == 0)
        def _p0():
            fn(0)

        @pl.when(par == 1)
        def _p1():
            fn(1)

    def start_gather(p, k, b):
        pltpu.async_copy(x2_hbm.at[isrc.at[p, k]], bufs[b], sg[b])

    def wait_gather(p, k, b):
        pltpu.make_async_copy(x2_hbm.at[isrc.at[p, k]], bufs[b], sg[b]).wait()

    def start_scatter(p, k, b):
        pltpu.async_copy(bufs[b], acc.at[idst.at[p, k]], ss[b], add=True)

    def wait_scatter(p, k, b):
        pltpu.make_async_copy(bufs[b], acc.at[idst.at[p, k]], ss[b]).wait()

    stage(0, 0, sync=True)
    start_gather(0, 0, 0)
    start_gather(0, 1, 1)

    @pl.loop(0, NBLK)
    def _blocks(g):
        p = lax.rem(g, 2)
        p1 = lax.rem(g + 1, 2)
        for k in range(G):
            b = k % 4
            wait_gather(p, k, b)
            start_scatter(p, k, b)
            if k >= 2:
                wait_scatter(p, k - 2, (k - 2) % 4)
            else:
                @pl.when(g >= 1)
                def _wprev():
                    wait_scatter(p1, G - 2 + k, (k + 2) % 4)
            if k <= G - 3:
                start_gather(p, k + 2, (k + 2) % 4)
            elif k == G - 2:
                @pl.when(g + 1 < NBLK)
                def _g6():
                    per_parity(g + 1, lambda q: wait_stage(g + 1, q))
                    start_gather(p1, 0, 0)
            else:
                @pl.when(g + 1 < NBLK)
                def _g7():
                    start_gather(p1, 1, 1)
            if k == 1:
                @pl.when(g + 1 < NBLK)
                def _st():
                    per_parity(g + 1, lambda q: stage(g + 1, q))

    wait_scatter((NBLK - 1) % 2, G - 2, (G - 2) % 4)
    wait_scatter((NBLK - 1) % 2, G - 1, (G - 1) % 4)
    plsc.subcore_barrier()
    pltpu.sync_copy(acc.at[pl.ds(tid * RPT, RPT)],
                    out_hbm.at[c, pl.ds(tid * RPT, RPT)])


@functools.cache
def _sc_edge():
    return pl.kernel(
        _sc_edge_body,
        out_type=jax.ShapeDtypeStruct((2, NP, H), jnp.float32),
        mesh=plsc.VectorSubcoreMesh(core_axis_name="c", subcore_axis_name="s"),
        scratch_types=[
            pltpu.VMEM((2, G, CHUNK), jnp.int32),
            pltpu.VMEM((2, G, CHUNK), jnp.int32),
            pltpu.VMEM((CHUNK, H), jnp.float32),
            pltpu.VMEM((CHUNK, H), jnp.float32),
            pltpu.VMEM((CHUNK, H), jnp.float32),
            pltpu.VMEM((CHUNK, H), jnp.float32),
            pltpu.VMEM_SHARED((NP, H), jnp.float32),
        ] + [pltpu.SemaphoreType.DMA] * 12,
        compiler_params=pltpu.CompilerParams(use_tc_tiling_on_sc=False),
    )


def kernel(x_n, edge_index, abs_level, rel_level, emb0, emb1, emb2,
           pi_w1, pi_b1, pi_w2, pi_b2,
           l0_w, l0_b, l0_wt, l0_bt, l0_ws, l0_bs,
           l1_w, l1_b, l1_wt, l1_bt, l1_ws, l1_bs,
           po_w1, po_b1, po_w2, po_b2):
    row = edge_index[0].astype(jnp.int32)
    col = edge_index[1].astype(jnp.int32)
    npad = E2P - 2 * E
    pad_idx = jnp.arange(npad, dtype=jnp.int32)
    pad_src = pad_idx % (2 * N)
    pad_dst = N + pad_idx % (NP - N)
    src = jnp.concatenate([col, row + N, pad_src]).reshape(NROW, CHUNK)
    dst = jnp.concatenate([row, col, pad_dst]).reshape(NROW, CHUNK)

    sc = _sc_edge()
    b2 = lambda v: v.reshape(1, H)
    h0, x2, s0 = _tc1(x_n.astype(jnp.int32), abs_level,
                      emb0, emb1, emb2,
                      pi_w1, b2(pi_b1), pi_w2, b2(pi_b2),
                      l0_w, b2(l0_b), l0_wt, b2(l0_bt), l0_ws, b2(l0_bs))
    p0 = sc(x2.reshape(2 * N, H), src, dst)
    h1, x2b, s1 = _tc2(p0, s0,
                       l1_w, b2(l1_b), l1_wt, b2(l1_bt), l1_ws, b2(l1_bs))
    p1 = sc(x2b.reshape(2 * N, H), src, dst)
    out = _tc3(p1, s1, h0, h1, po_w1, b2(po_b1), po_w2, b2(po_b2))
    return out


# speedup vs baseline: 1.1144x; 1.1144x over previous
import functools
import math

import jax
import jax.numpy as jnp
from jax import lax
from jax.experimental import pallas as pl
from jax.experimental.pallas import tpu as pltpu
from jax.experimental.pallas import tpu_sc as plsc

N = 10000
E = 320000
H = 128
BLK = 2000
GRID = N // BLK

CHUNK = 128
G = 8
NW = 32
CPW = 160
NBLK = CPW // G
E2P = NW * CPW * CHUNK
NROW = E2P // CHUNK
NP = 10240
RPT = NP // 16


def _gelu(x):
    return 0.5 * x * (1.0 + lax.erf(x * (1.0 / math.sqrt(2.0))))


def _dense(h, w, b):
    return jnp.dot(h, w[...], preferred_element_type=jnp.float32) + b[...]


def _tc1_body(x_ref, al_ref, e0, e1, e2, pw1, pb1, pw2, pb2,
              w, b, wt, bt, ws, bs, h0_out, x2_out, s_out):
    x = x_ref[...]
    oh0 = (x[:, 0:1] == lax.broadcasted_iota(jnp.int32, (BLK, 16), 1))
    oh1 = (x[:, 1:2] == lax.broadcasted_iota(jnp.int32, (BLK, 8), 1))
    oh2 = (x[:, 2:3] == lax.broadcasted_iota(jnp.int32, (BLK, 4), 1))
    he0 = jnp.dot(oh0.astype(jnp.float32), e0[...],
                  preferred_element_type=jnp.float32)
    he1 = jnp.dot(oh1.astype(jnp.float32), e1[...],
                  preferred_element_type=jnp.float32)
    he2 = jnp.dot(oh2.astype(jnp.float32), e2[...],
                  preferred_element_type=jnp.float32)
    al = al_ref[...]
    k2 = lax.broadcasted_iota(jnp.int32, (1, 16), 1).astype(jnp.float32) * 2.0
    div = jnp.exp(k2 * (-math.log(10000.0) / 32.0))
    ang = al * div
    h = jnp.concatenate([he0, he1, he2, jnp.sin(ang), jnp.cos(ang)], axis=1)
    h = _gelu(_dense(h, pw1, pb1))
    h = _dense(h, pw2, pb2)
    h0_out[...] = h
    x2_out[0] = _dense(h, w, b)
    x2_out[1] = _dense(h, wt, bt)
    s_out[...] = _dense(h, ws, bs)


def _tc2_body(p_ref, s_ref, w, b, wt, bt, ws, bs, h_out, x2_out, s_out):
    h = _gelu(p_ref[0] + p_ref[1] + s_ref[...])
    h_out[...] = h
    x2_out[0] = _dense(h, w, b)
    x2_out[1] = _dense(h, wt, bt)
    s_out[...] = _dense(h, ws, bs)


def _tc3_body(p_ref, s_ref, h0_ref, h1_ref, pw1, pb1, pw2, pb2, out_ref):
    h2 = _gelu(p_ref[0] + p_ref[1] + s_ref[...])
    t = (jnp.dot(h0_ref[...], pw1[0:H], preferred_element_type=jnp.float32)
         + jnp.dot(h1_ref[...], pw1[H:2 * H], preferred_element_type=jnp.float32)
         + jnp.dot(h2, pw1[2 * H:3 * H], preferred_element_type=jnp.float32)
         + pb1[...])
    out_ref[...] = (jnp.dot(_gelu(t), pw2[...],
                            preferred_element_type=jnp.float32) + pb2[...])


def _row_spec():
    return pl.BlockSpec((BLK, H), lambda i: (i, 0))


def _full(shape):
    return pl.BlockSpec(shape, lambda i: tuple(0 for _ in shape))


_W = _full((H, H))
_B = _full((1, H))
_P2 = pl.BlockSpec((2, BLK, H), lambda i: (0, i, 0))
_X2_SPEC = pl.BlockSpec((2, BLK, H), lambda i: (0, i, 0))
_X2 = jax.ShapeDtypeStruct((2, N, H), jnp.float32)

_tc1 = pl.pallas_call(
    _tc1_body,
    grid=(GRID,),
    in_specs=[pl.BlockSpec((BLK, 3), lambda i: (i, 0)),
              pl.BlockSpec((BLK, 1), lambda i: (i, 0)),
              _full((16, 32)), _full((8, 32)), _full((4, 32)),
              _W, _B, _W, _B,
              _W, _B, _W, _B, _W, _B],
    out_specs=[_row_spec(), _X2_SPEC, _row_spec()],
    out_shape=[jax.ShapeDtypeStruct((N, H), jnp.float32), _X2,
               jax.ShapeDtypeStruct((N, H), jnp.float32)],
)

_tc2 = pl.pallas_call(
    _tc2_body,
    grid=(GRID,),
    in_specs=[_P2, _row_spec(),
              _W, _B, _W, _B, _W, _B],
    out_specs=[_row_spec(), _X2_SPEC, _row_spec()],
    out_shape=[jax.ShapeDtypeStruct((N, H), jnp.float32), _X2,
               jax.ShapeDtypeStruct((N, H), jnp.float32)],
)

_tc3 = pl.pallas_call(
    _tc3_body,
    grid=(GRID,),
    in_specs=[_P2, _row_spec(), _row_spec(), _row_spec(),
              _full((3 * H, H)), _B, _W, _B],
    out_specs=_row_spec(),
    out_shape=jax.ShapeDtypeStruct((N, H), jnp.float32),
)


def _sc_edge_body(x2_hbm, src_hbm, dst_hbm, out_hbm,
                  isrc, idst, buf0, buf1, acc,
                  sg0, sg1, si0, si1, sj0, sj1):
    c = lax.axis_index("c")
    tid = lax.axis_index("s")
    wid = tid * 2 + c
    base = wid * CPW

    bufs = (buf0, buf1)
    sg = (sg0, sg1)
    si = (si0, si1)
    sj = (sj0, sj1)

    @pl.loop(0, CHUNK)
    def _zero_rows(i):
        for k in range(H // 16):
            buf0[i, pl.ds(k * 16, 16)] = jnp.zeros((16,), jnp.float32)

    for r in range(RPT // CHUNK):
        pltpu.sync_copy(buf0, acc.at[pl.ds((tid * (RPT // CHUNK) + r) * CHUNK,
                                           CHUNK)])
    plsc.subcore_barrier()

    def stage(g, p, sync=False):
        rows = pl.ds(base + g * G, G)
        if sync:
            pltpu.sync_copy(src_hbm.at[rows], isrc.at[p])
            pltpu.sync_copy(dst_hbm.at[rows], idst.at[p])
        else:
            pltpu.async_copy(src_hbm.at[rows], isrc.at[p], si[p])
            pltpu.async_copy(dst_hbm.at[rows], idst.at[p], sj[p])

    def wait_stage(g, p):
        rows = pl.ds(base + g * G, G)
        pltpu.make_async_copy(src_hbm.at[rows], isrc.at[p], si[p]).wait()
        pltpu.make_async_copy(dst_hbm.at[rows], idst.at[p], sj[p]).wait()

    def per_parity(g1, fn):
        par = lax.rem(g1, 2)

        @pl.when(par == 0)
        def _p0():
            fn(0)

        @pl.when(par == 1)
        def _p1():
            fn(1)

    def start_gather(p, k, b):
        pltpu.async_copy(x2_hbm.at[isrc.at[p, k]], bufs[b], sg[b])

    def wait_gather(p, k, b):
        pltpu.make_async_copy(x2_hbm.at[isrc.at[p, k]], bufs[b], sg[b]).wait()

    def sync_scatter(p, k, b):
        pltpu.sync_copy(bufs[b], acc.at[idst.at[p, k]], add=True)

    stage(0, 0, sync=True)
    start_gather(0, 0, 0)

    @pl.loop(0, NBLK)
    def _blocks(g):
        p = lax.rem(g, 2)
        p1 = lax.rem(g + 1, 2)
        for k in range(G):
            jj_par = k % 2
            if k < G - 1:
                start_gather(p, k + 1, 1 - jj_par)
            else:
                @pl.when(g + 1 < NBLK)
                def _gnext():
                    per_parity(g + 1, lambda q: wait_stage(g + 1, q))
                    start_gather(p1, 0, 1 - jj_par)
            if k == 1:
                @pl.when(g + 1 < NBLK)
                def _st():
                    per_parity(g + 1, lambda q: stage(g + 1, q))
            wait_gather(p, k, jj_par)
            sync_scatter(p, k, jj_par)
    plsc.subcore_barrier()
    pltpu.sync_copy(acc.at[pl.ds(tid * RPT, RPT)],
                    out_hbm.at[c, pl.ds(tid * RPT, RPT)])


@functools.cache
def _sc_edge():
    return pl.kernel(
        _sc_edge_body,
        out_type=jax.ShapeDtypeStruct((2, NP, H), jnp.float32),
        mesh=plsc.VectorSubcoreMesh(core_axis_name="c", subcore_axis_name="s"),
        scratch_types=[
            pltpu.VMEM((2, G, CHUNK), jnp.int32),
            pltpu.VMEM((2, G, CHUNK), jnp.int32),
            pltpu.VMEM((CHUNK, H), jnp.float32),
            pltpu.VMEM((CHUNK, H), jnp.float32),
            pltpu.VMEM_SHARED((NP, H), jnp.float32),
            pltpu.SemaphoreType.DMA,
            pltpu.SemaphoreType.DMA,
            pltpu.SemaphoreType.DMA,
            pltpu.SemaphoreType.DMA,
            pltpu.SemaphoreType.DMA,
            pltpu.SemaphoreType.DMA,
        ],
        compiler_params=pltpu.CompilerParams(use_tc_tiling_on_sc=False),
    )


def kernel(x_n, edge_index, abs_level, rel_level, emb0, emb1, emb2,
           pi_w1, pi_b1, pi_w2, pi_b2,
           l0_w, l0_b, l0_wt, l0_bt, l0_ws, l0_bs,
           l1_w, l1_b, l1_wt, l1_bt, l1_ws, l1_bs,
           po_w1, po_b1, po_w2, po_b2):
    row = edge_index[0].astype(jnp.int32)
    col = edge_index[1].astype(jnp.int32)
    npad = E2P - 2 * E
    pad_idx = jnp.arange(npad, dtype=jnp.int32)
    pad_src = pad_idx % (2 * N)
    pad_dst = N + pad_idx % (NP - N)
    src = jnp.concatenate([col, row + N, pad_src]).reshape(NROW, CHUNK)
    dst = jnp.concatenate([row, col, pad_dst]).reshape(NROW, CHUNK)

    sc = _sc_edge()
    b2 = lambda v: v.reshape(1, H)
    h0, x2, s0 = _tc1(x_n.astype(jnp.int32), abs_level,
                      emb0, emb1, emb2,
                      pi_w1, b2(pi_b1), pi_w2, b2(pi_b2),
                      l0_w, b2(l0_b), l0_wt, b2(l0_bt), l0_ws, b2(l0_bs))
    p0 = sc(x2.reshape(2 * N, H), src, dst)
    h1, x2b, s1 = _tc2(p0, s0,
                       l1_w, b2(l1_b), l1_wt, b2(l1_bt), l1_ws, b2(l1_bs))
    p1 = sc(x2b.reshape(2 * N, H), src, dst)
    out = _tc3(p1, s1, h0, h1, po_w1, b2(po_b1), po_w2, b2(po_b2))
    return out
